# Initial kernel scaffold; baseline (speedup 1.0000x reference)
import jax
import jax.numpy as jnp
from jax.experimental import pallas as pl


def kernel(z, embedding, cluster_size, embedding_mean):
    raise NotImplementedError("write your pallas kernel here")


# speedup vs baseline: 1.2445x; 1.2445x over previous
import functools

import jax
import jax.numpy as jnp
from jax.experimental import pallas as pl
from jax.experimental.pallas import tpu as pltpu

B, T = 16, 1024
D = 256
K = 8192
N = B * T
DECAY = 0.99
OMD = 1.0 - DECAY
EPS = 1e-05
KEPS = K * EPS

NT_A = 1024
KT_A = 1408
KPAD = 6 * KT_A


def _argmin_body(z2_ref, c2_ref, z_ref, e_ref, idx_ref,
                 iv_ref, ii_ref, ov_ref, oi_ref):
    j = pl.program_id(1)
    mm = jax.lax.dot_general(
        z_ref[...], e_ref[...], (((1,), (0,)), ((), ())),
        preferred_element_type=jnp.float32)
    dist = (z2_ref[...] - 2.0 * mm) + c2_ref[...]
    lv = jnp.min(dist, axis=1, keepdims=True)
    iota = jax.lax.broadcasted_iota(jnp.int32, (NT_A, KT_A), 1)
    cand = jnp.where(dist == lv, iota, jnp.int32(2 ** 30))
    li = jnp.min(cand, axis=1, keepdims=True) + j * KT_A

    @pl.when(j % 2 == 0)
    def _():
        iv_ref[...] = lv
        ii_ref[...] = li

    @pl.when(j % 2 == 1)
    def _():
        upd = lv < iv_ref[...]
        iv = jnp.where(upd, lv, iv_ref[...])
        ii = jnp.where(upd, li, ii_ref[...])
        inf = jnp.full((NT_A, 1), jnp.inf, jnp.float32)
        prev_ov = jnp.where(j == 1, inf, ov_ref[...])
        prev_oi = jnp.where(j == 1, jnp.zeros((NT_A, 1), jnp.int32),
                            oi_ref[...])
        upd2 = iv < prev_ov
        ivr = iv.astype(jnp.bfloat16).astype(jnp.float32)
        ov_ref[...] = jnp.where(upd2, ivr, prev_ov)
        oi_ref[...] = jnp.where(upd2, ii, prev_oi)

    @pl.when(j == 5)
    def _():
        idx_ref[...] = oi_ref[...]


def _argmin_call(z_flat, z2, c2_pad, emb_pad):
    return pl.pallas_call(
        _argmin_body,
        grid=(N // NT_A, KPAD // KT_A),
        in_specs=[
            pl.BlockSpec((NT_A, 1), lambda i, j: (i, 0)),
            pl.BlockSpec((1, KT_A), lambda i, j: (0, j)),
            pl.BlockSpec((NT_A, D), lambda i, j: (i, 0)),
            pl.BlockSpec((D, KT_A), lambda i, j: (0, j)),
        ],
        out_specs=pl.BlockSpec((NT_A, 1), lambda i, j: (i, 0)),
        out_shape=jax.ShapeDtypeStruct((N, 1), jnp.int32),
        scratch_shapes=[
            pltpu.VMEM((NT_A, 1), jnp.float32),
            pltpu.VMEM((NT_A, 1), jnp.int32),
            pltpu.VMEM((NT_A, 1), jnp.float32),
            pltpu.VMEM((NT_A, 1), jnp.int32),
        ],
        compiler_params=pltpu.CompilerParams(
            dimension_semantics=("arbitrary", "arbitrary")),
    )(z2, c2_pad, z_flat, emb_pad)


NT_B = 1024
KC_B = 1024


def _scatter_body(idx_ref, z_ref, sums_ref, counts_ref):
    n = pl.program_id(0)

    @pl.when(n == 0)
    def _():
        sums_ref[...] = jnp.zeros_like(sums_ref)
        counts_ref[...] = jnp.zeros_like(counts_ref)

    idx = idx_ref[...]
    zb = z_ref[...]
    for kc in range(K // KC_B):
        i0 = kc * KC_B
        iota = jax.lax.broadcasted_iota(jnp.int32, (NT_B, KC_B), 1) + i0
        oh = (idx == iota).astype(jnp.float32)
        ps = jax.lax.dot_general(
            oh, zb, (((0,), (0,)), ((), ())),
            preferred_element_type=jnp.float32)
        sums_ref[pl.ds(i0, KC_B), :] += ps
        counts_ref[0, pl.ds(i0, KC_B)] += jnp.sum(oh, axis=0)


def _scatter_call(idx, z_flat):
    return pl.pallas_call(
        _scatter_body,
        grid=(N // NT_B,),
        in_specs=[
            pl.BlockSpec((NT_B, 1), lambda n: (n, 0)),
            pl.BlockSpec((NT_B, D), lambda n: (n, 0)),
        ],
        out_specs=[
            pl.BlockSpec((K, D), lambda n: (0, 0)),
            pl.BlockSpec((1, K), lambda n: (0, 0)),
        ],
        out_shape=[
            jax.ShapeDtypeStruct((K, D), jnp.float32),
            jax.ShapeDtypeStruct((1, K), jnp.float32),
        ],
        compiler_params=pltpu.CompilerParams(
            dimension_semantics=("arbitrary",)),
    )(idx, z_flat)


KT_C = 512


def _norm_body(cs_ref, cnt_ref, em_ref, sums_ref, ne_ref):
    k = pl.program_id(0)
    ncs_full = cs_ref[...] * DECAY + cnt_ref[...] * OMD
    n = jnp.sum(ncs_full)
    ncs = (cs_ref[:, pl.ds(k * KT_C, KT_C)] * DECAY
           + cnt_ref[:, pl.ds(k * KT_C, KT_C)] * OMD)
    cs = (ncs + EPS) / (n + KEPS) * n
    em_t = jnp.swapaxes(em_ref[...], 0, 1)
    cs_col = jnp.swapaxes(cs, 0, 1)
    ne_ref[...] = (em_t * DECAY + sums_ref[...] * OMD) / cs_col


def _norm_call(cluster_row, counts_row, emb_mean, sums):
    return pl.pallas_call(
        _norm_body,
        grid=(K // KT_C,),
        in_specs=[
            pl.BlockSpec((1, K), lambda k: (0, 0)),
            pl.BlockSpec((1, K), lambda k: (0, 0)),
            pl.BlockSpec((D, KT_C), lambda k: (0, k)),
            pl.BlockSpec((KT_C, D), lambda k: (k, 0)),
        ],
        out_specs=pl.BlockSpec((KT_C, D), lambda k: (k, 0)),
        out_shape=jax.ShapeDtypeStruct((K, D), jnp.float32),
        compiler_params=pltpu.CompilerParams(
            dimension_semantics=("arbitrary",)),
    )(cluster_row, counts_row, emb_mean, sums)


NT_D = 1024
KC_D = 1024


def _gather_body(idx_ref, ne_ref, zq_ref):
    idx = idx_ref[...]
    acc = jnp.zeros((NT_D, D), jnp.float32)
    for kc in range(K // KC_D):
        i0 = kc * KC_D
        iota = jax.lax.broadcasted_iota(jnp.int32, (NT_D, KC_D), 1) + i0
        oh = (idx == iota).astype(jnp.float32)
        acc = acc + jax.lax.dot_general(
            oh, ne_ref[pl.ds(i0, KC_D), :], (((1,), (0,)), ((), ())),
            preferred_element_type=jnp.float32)
    zq_ref[...] = acc


def _gather_call(idx, ne):
    return pl.pallas_call(
        _gather_body,
        grid=(N // NT_D,),
        in_specs=[
            pl.BlockSpec((NT_D, 1), lambda n: (n, 0)),
            pl.BlockSpec((K, D), lambda n: (0, 0)),
        ],
        out_specs=pl.BlockSpec((NT_D, D), lambda n: (n, 0)),
        out_shape=jax.ShapeDtypeStruct((N, D), jnp.float32),
        compiler_params=pltpu.CompilerParams(
            dimension_semantics=("arbitrary",)),
    )(idx, ne)


def _st_body(z_ref, zq_ref, st_ref, loss_ref):
    d = zq_ref[...] - z_ref[...]
    st_ref[...] = z_ref[...] + d
    loss_ref[...] = (jnp.zeros((1, 1, 1), jnp.float32)
                     + jnp.sum(d * d) / (T * D))


def _st_call(z_flat, zq):
    return pl.pallas_call(
        _st_body,
        grid=(B,),
        in_specs=[
            pl.BlockSpec((T, D), lambda b: (b, 0)),
            pl.BlockSpec((T, D), lambda b: (b, 0)),
        ],
        out_specs=[
            pl.BlockSpec((T, D), lambda b: (b, 0)),
            pl.BlockSpec((1, 1, 1), lambda b: (b, 0, 0)),
        ],
        out_shape=[
            jax.ShapeDtypeStruct((N, D), jnp.float32),
            jax.ShapeDtypeStruct((B, 1, 1), jnp.float32),
        ],
        compiler_params=pltpu.CompilerParams(
            dimension_semantics=("arbitrary",)),
    )(z_flat, zq)


def kernel(z, embedding, cluster_size, embedding_mean):
    z_flat = z.reshape(N, D)
    z2 = (z_flat ** 2).sum(1, keepdims=True)
    codebook = embedding.T
    c2 = (codebook ** 2).sum(1, keepdims=True).T
    emb_pad = jnp.pad(embedding, ((0, 0), (0, KPAD - K)))
    c2_pad = jnp.pad(c2, ((0, 0), (0, KPAD - K)), constant_values=1e30)

    idx = _argmin_call(z_flat, z2, c2_pad, emb_pad)
    sums, counts = _scatter_call(idx, z_flat)
    ne = _norm_call(cluster_size.reshape(1, K), counts,
                    embedding_mean, sums)
    zq = _gather_call(idx, ne)
    st, loss = _st_call(z_flat, zq)

    z_q_st = st.reshape(B, T, D)
    commitment_loss = loss.reshape(B)
    codebook_loss = jnp.zeros((B,), dtype=z.dtype)
    indices = idx.reshape(B, T)
    return (z_q_st, commitment_loss, codebook_loss, indices)


# speedup vs baseline: 1.3217x; 1.0620x over previous
import functools

import jax
import jax.numpy as jnp
from jax import lax
from jax.experimental import pallas as pl
from jax.experimental.pallas import tpu as pltpu
from jax.experimental.pallas import tpu_sc as plsc

B, T = 16, 1024
D = 256
K = 8192
N = B * T
DECAY = 0.99
OMD = 1.0 - DECAY
EPS = 1e-05
KEPS = K * EPS

NT_A = 1024
KT_A = 1408
KPAD = 6 * KT_A


def _argmin_body(z2_ref, c2_ref, z_ref, e_ref, idx_ref,
                 iv_ref, ii_ref, ov_ref, oi_ref):
    j = pl.program_id(1)
    mm = jax.lax.dot_general(
        z_ref[...], e_ref[...], (((1,), (0,)), ((), ())),
        preferred_element_type=jnp.float32)
    dist = (z2_ref[...] - 2.0 * mm) + c2_ref[...]
    lv = jnp.min(dist, axis=1, keepdims=True)
    iota = jax.lax.broadcasted_iota(jnp.int32, (NT_A, KT_A), 1)
    cand = jnp.where(dist == lv, iota, jnp.int32(2 ** 30))
    li = jnp.min(cand, axis=1, keepdims=True) + j * KT_A

    @pl.when(j % 2 == 0)
    def _():
        iv_ref[...] = lv
        ii_ref[...] = li

    @pl.when(j % 2 == 1)
    def _():
        upd = lv < iv_ref[...]
        iv = jnp.where(upd, lv, iv_ref[...])
        ii = jnp.where(upd, li, ii_ref[...])
        inf = jnp.full((NT_A, 1), jnp.inf, jnp.float32)
        prev_ov = jnp.where(j == 1, inf, ov_ref[...])
        prev_oi = jnp.where(j == 1, jnp.zeros((NT_A, 1), jnp.int32),
                            oi_ref[...])
        upd2 = iv < prev_ov
        ivr = iv.astype(jnp.bfloat16).astype(jnp.float32)
        ov_ref[...] = jnp.where(upd2, ivr, prev_ov)
        oi_ref[...] = jnp.where(upd2, ii, prev_oi)

    @pl.when(j == 5)
    def _():
        idx_ref[...] = oi_ref[...]


def _argmin_call(z_flat, z2, c2_pad, emb_pad):
    return pl.pallas_call(
        _argmin_body,
        grid=(N // NT_A, KPAD // KT_A),
        in_specs=[
            pl.BlockSpec((NT_A, 1), lambda i, j: (i, 0)),
            pl.BlockSpec((1, KT_A), lambda i, j: (0, j)),
            pl.BlockSpec((NT_A, D), lambda i, j: (i, 0)),
            pl.BlockSpec((D, KT_A), lambda i, j: (0, j)),
        ],
        out_specs=pl.BlockSpec((NT_A, 1), lambda i, j: (i, 0)),
        out_shape=jax.ShapeDtypeStruct((N, 1), jnp.int32),
        scratch_shapes=[
            pltpu.VMEM((NT_A, 1), jnp.float32),
            pltpu.VMEM((NT_A, 1), jnp.int32),
            pltpu.VMEM((NT_A, 1), jnp.float32),
            pltpu.VMEM((NT_A, 1), jnp.int32),
        ],
        compiler_params=pltpu.CompilerParams(
            dimension_semantics=("arbitrary", "arbitrary")),
    )(z2, c2_pad, z_flat, emb_pad)


NT_B = 1024
KC_B = 1024


def _scatter_body(idx_ref, z_ref, sums_ref, counts_ref):
    n = pl.program_id(0)

    @pl.when(n == 0)
    def _():
        sums_ref[...] = jnp.zeros_like(sums_ref)
        counts_ref[...] = jnp.zeros_like(counts_ref)

    idx = idx_ref[...]
    zb = z_ref[...]
    for kc in range(K // KC_B):
        i0 = kc * KC_B
        iota = jax.lax.broadcasted_iota(jnp.int32, (NT_B, KC_B), 1) + i0
        oh = (idx == iota).astype(jnp.float32)
        ps = jax.lax.dot_general(
            oh, zb, (((0,), (0,)), ((), ())),
            preferred_element_type=jnp.float32)
        sums_ref[pl.ds(i0, KC_B), :] += ps
        counts_ref[0, pl.ds(i0, KC_B)] += jnp.sum(oh, axis=0)


def _scatter_call(idx, z_flat):
    return pl.pallas_call(
        _scatter_body,
        grid=(N // NT_B,),
        in_specs=[
            pl.BlockSpec((NT_B, 1), lambda n: (n, 0)),
            pl.BlockSpec((NT_B, D), lambda n: (n, 0)),
        ],
        out_specs=[
            pl.BlockSpec((K, D), lambda n: (0, 0)),
            pl.BlockSpec((1, K), lambda n: (0, 0)),
        ],
        out_shape=[
            jax.ShapeDtypeStruct((K, D), jnp.float32),
            jax.ShapeDtypeStruct((1, K), jnp.float32),
        ],
        compiler_params=pltpu.CompilerParams(
            dimension_semantics=("arbitrary",)),
    )(idx, z_flat)


KT_C = 512


def _norm_body(cs_ref, cnt_ref, em_ref, sums_ref, ne_ref):
    k = pl.program_id(0)
    ncs_full = cs_ref[...] * DECAY + cnt_ref[...] * OMD
    n = jnp.sum(ncs_full)
    ncs = (cs_ref[:, pl.ds(k * KT_C, KT_C)] * DECAY
           + cnt_ref[:, pl.ds(k * KT_C, KT_C)] * OMD)
    cs = (ncs + EPS) / (n + KEPS) * n
    em_t = jnp.swapaxes(em_ref[...], 0, 1)
    cs_col = jnp.swapaxes(cs, 0, 1)
    ne_ref[...] = (em_t * DECAY + sums_ref[...] * OMD) / cs_col


def _norm_call(cluster_row, counts_row, emb_mean, sums):
    return pl.pallas_call(
        _norm_body,
        grid=(K // KT_C,),
        in_specs=[
            pl.BlockSpec((1, K), lambda k: (0, 0)),
            pl.BlockSpec((1, K), lambda k: (0, 0)),
            pl.BlockSpec((D, KT_C), lambda k: (0, k)),
            pl.BlockSpec((KT_C, D), lambda k: (k, 0)),
        ],
        out_specs=pl.BlockSpec((KT_C, D), lambda k: (k, 0)),
        out_shape=jax.ShapeDtypeStruct((K, D), jnp.float32),
        compiler_params=pltpu.CompilerParams(
            dimension_semantics=("arbitrary",)),
    )(cluster_row, counts_row, emb_mean, sums)


SC_NC, SC_NS = 2, 16
SC_NW = SC_NC * SC_NS
SC_BPW = N // SC_NW
SC_CH = 256


@functools.partial(
    pl.kernel,
    mesh=plsc.VectorSubcoreMesh(core_axis_name="c", subcore_axis_name="s"),
    out_type=jax.ShapeDtypeStruct((N, D), jnp.float32),
    scratch_types=[
        pltpu.VMEM((SC_CH,), jnp.int32),
        pltpu.VMEM((SC_CH, D), jnp.float32),
        pltpu.SemaphoreType.DMA,
    ],
)
def _sc_gather(ne_hbm, idx_hbm, out_hbm, idx_v, rows_v, sem):
    wid = lax.axis_index("s") * SC_NC + lax.axis_index("c")
    for ch in range(SC_BPW // SC_CH):
        base = wid * SC_BPW + ch * SC_CH
        pltpu.sync_copy(idx_hbm.at[pl.ds(base, SC_CH)], idx_v)
        pltpu.async_copy(ne_hbm.at[idx_v], rows_v, sem).wait()
        pltpu.sync_copy(rows_v, out_hbm.at[pl.ds(base, SC_CH)])


def _gather_call(idx, ne):
    return _sc_gather(ne, idx.reshape(N))


def _st_body(z_ref, zq_ref, st_ref, loss_ref):
    d = zq_ref[...] - z_ref[...]
    st_ref[...] = z_ref[...] + d
    loss_ref[...] = (jnp.zeros((1, 1, 1), jnp.float32)
                     + jnp.sum(d * d) / (T * D))


def _st_call(z_flat, zq):
    return pl.pallas_call(
        _st_body,
        grid=(B,),
        in_specs=[
            pl.BlockSpec((T, D), lambda b: (b, 0)),
            pl.BlockSpec((T, D), lambda b: (b, 0)),
        ],
        out_specs=[
            pl.BlockSpec((T, D), lambda b: (b, 0)),
            pl.BlockSpec((1, 1, 1), lambda b: (b, 0, 0)),
        ],
        out_shape=[
            jax.ShapeDtypeStruct((N, D), jnp.float32),
            jax.ShapeDtypeStruct((B, 1, 1), jnp.float32),
        ],
        compiler_params=pltpu.CompilerParams(
            dimension_semantics=("arbitrary",)),
    )(z_flat, zq)


def kernel(z, embedding, cluster_size, embedding_mean):
    z_flat = z.reshape(N, D)
    z2 = (z_flat ** 2).sum(1, keepdims=True)
    codebook = embedding.T
    c2 = (codebook ** 2).sum(1, keepdims=True).T
    emb_pad = jnp.pad(embedding, ((0, 0), (0, KPAD - K)))
    c2_pad = jnp.pad(c2, ((0, 0), (0, KPAD - K)), constant_values=1e30)

    idx = _argmin_call(z_flat, z2, c2_pad, emb_pad)
    sums, counts = _scatter_call(idx, z_flat)
    ne = _norm_call(cluster_size.reshape(1, K), counts,
                    embedding_mean, sums)
    zq = _gather_call(idx, ne)
    st, loss = _st_call(z_flat, zq)

    z_q_st = st.reshape(B, T, D)
    commitment_loss = loss.reshape(B)
    codebook_loss = jnp.zeros((B,), dtype=z.dtype)
    indices = idx.reshape(B, T)
    return (z_q_st, commitment_loss, codebook_loss, indices)
